# Initial kernel scaffold; baseline (speedup 1.0000x reference)
import jax
import jax.numpy as jnp
from jax.experimental import pallas as pl


def kernel(x, w1c, b1c, w2c, b2c, wf1, bf1, wf2, bf2, wf3, bf3):
    raise NotImplementedError("write your pallas kernel here")


# speedup vs baseline: 35.5883x; 35.5883x over previous
import functools

import jax
import jax.numpy as jnp
from jax.experimental import pallas as pl
from jax.experimental.pallas import tpu as pltpu


def _round_up(x, m):
    return (x + m - 1) // m * m


_W_IN1, _W_IN2 = 56, 156
_OW1, _OW2 = 26, 11
_C1, _C2 = 6, 16
_N1, _N2 = _C1 * _OW1, _C2 * _OW2
_R1, _R2 = 12, 4


def _fused_kernel(xt_ref, m1a_ref, m1b_ref, m2a_ref, m2b_ref, bc1_ref,
                  bc2_ref, w1_ref, b1_ref, w2_ref, b2_ref, w3_ref, b3_ref,
                  o_ref):
    y1 = []
    for r in range(_R1):
        acc = None
        for a in range(2):
            p = jnp.concatenate(
                [xt_ref[2 * r + a + kh] for kh in range(5)], axis=1)
            for m_ref in (m1a_ref, m1b_ref):
                z = jnp.dot(p, m_ref[...],
                            preferred_element_type=jnp.float32)
                acc = z if acc is None else jnp.maximum(acc, z)
        y1.append(jnp.maximum(acc + bc1_ref[...], 0.0))

    y2 = []
    for r in range(_R2):
        acc = None
        for a in range(2):
            p = jnp.concatenate(
                [y1[2 * r + a + kh] for kh in range(5)], axis=1)
            for m_ref in (m2a_ref, m2b_ref):
                z = jnp.dot(p, m_ref[...],
                            preferred_element_type=jnp.float32)
                acc = z if acc is None else jnp.maximum(acc, z)
        y2.append(jnp.maximum(acc + bc2_ref[...], 0.0))

    feat = jnp.concatenate(y2, axis=1)
    h = jnp.dot(feat, w1_ref[...], preferred_element_type=jnp.float32)
    h = jnp.maximum(h + b1_ref[...], 0.0)
    h = jnp.dot(h, w2_ref[...], preferred_element_type=jnp.float32)
    h = jnp.maximum(h + b2_ref[...], 0.0)
    z = jnp.dot(h, w3_ref[...], preferred_element_type=jnp.float32)
    z = z + b3_ref[...]
    z = z - jnp.max(z, axis=-1, keepdims=True)
    e = jnp.exp(z)
    o_ref[...] = (e / jnp.sum(e, axis=-1, keepdims=True))[:, :o_ref.shape[1]]


def _banded_conv_matrix(w, c_in, c_out, w_in, ow, b):
    kh, c, kw, co, owi = jnp.meshgrid(
        jnp.arange(5), jnp.arange(c_in), jnp.arange(5), jnp.arange(c_out),
        jnp.arange(ow), indexing="ij")
    rows = kh * (c_in * w_in) + c * w_in + 2 * owi + b + kw
    cols = co * ow + owi
    vals = w[co, c, kh, kw]
    m = jnp.zeros((5 * c_in * w_in, c_out * ow), jnp.float32)
    return m.at[rows.ravel(), cols.ravel()].set(vals.ravel())


@functools.partial(jax.jit, static_argnames=("interpret",))
def _forward(x, w1c, b1c, w2c, b2c, wf1, bf1, wf2, bf2, wf3, bf3,
             interpret=False):
    n = x.shape[0]
    bt = 256 if n >= 256 else _round_up(n, 8)
    n_pad = _round_up(n, bt)
    xs = jnp.pad(x[:, 0], ((0, n_pad - n), (0, 0), (0, 0)))
    xt = xs.transpose(1, 0, 2)

    w1 = w1c.reshape(_C1, 1, 5, 5).astype(jnp.float32)
    w2 = w2c.astype(jnp.float32)
    m1a = _banded_conv_matrix(w1, 1, _C1, _W_IN1, _OW1, 0)
    m1b = _banded_conv_matrix(w1, 1, _C1, _W_IN1, _OW1, 1)
    m2a = _banded_conv_matrix(w2, _C1, _C2, _OW1, _OW2, 0)
    m2b = _banded_conv_matrix(w2, _C1, _C2, _OW1, _OW2, 1)
    bc1 = jnp.repeat(b1c, _OW1).reshape(1, _N1)
    bc2 = jnp.repeat(b2c, _OW2).reshape(1, _N2)

    w1s = wf1.reshape(120, _C2, _R2, _OW2).transpose(2, 1, 3, 0)
    w1s = jnp.pad(w1s.reshape(_R2 * _N2, 120), ((0, 0), (0, 8)))
    b1f = jnp.pad(bf1, (0, 8)).reshape(1, 128)
    w2p = jnp.pad(wf2.T, ((0, 8), (0, 44)))
    b2f = jnp.pad(bf2, (0, 44)).reshape(1, 128)
    n_cls = wf3.shape[0]
    w3p = jnp.pad(wf3.T, ((0, 44), (0, 128 - n_cls)))
    b3f = jnp.concatenate(
        [bf3, jnp.full((128 - n_cls,), -jnp.inf, jnp.float32)]).reshape(1, 128)

    grid = (n_pad // bt,)
    flops = 2 * n_pad * (_R1 * 4 * 280 * _N1 + _R2 * 4 * 780 * _N2
                         + 704 * 128 + 2 * 128 * 128)
    bytes_accessed = 4 * (28 * n_pad * 56 + n_pad * n_cls + 300000)

    out = pl.pallas_call(
        _fused_kernel,
        out_shape=jax.ShapeDtypeStruct((n_pad, n_cls), jnp.float32),
        grid=grid,
        in_specs=[
            pl.BlockSpec((28, bt, 56), lambda i: (0, i, 0)),
            pl.BlockSpec((280, _N1), lambda i: (0, 0)),
            pl.BlockSpec((280, _N1), lambda i: (0, 0)),
            pl.BlockSpec((780, _N2), lambda i: (0, 0)),
            pl.BlockSpec((780, _N2), lambda i: (0, 0)),
            pl.BlockSpec((1, _N1), lambda i: (0, 0)),
            pl.BlockSpec((1, _N2), lambda i: (0, 0)),
            pl.BlockSpec((_R2 * _N2, 128), lambda i: (0, 0)),
            pl.BlockSpec((1, 128), lambda i: (0, 0)),
            pl.BlockSpec((128, 128), lambda i: (0, 0)),
            pl.BlockSpec((1, 128), lambda i: (0, 0)),
            pl.BlockSpec((128, 128), lambda i: (0, 0)),
            pl.BlockSpec((1, 128), lambda i: (0, 0)),
        ],
        out_specs=pl.BlockSpec((bt, n_cls), lambda i: (i, 0)),
        compiler_params=pltpu.CompilerParams(
            dimension_semantics=("parallel",),
            vmem_limit_bytes=64 * 1024 * 1024),
        cost_estimate=pl.CostEstimate(flops=flops, transcendentals=n_pad * 128,
                                      bytes_accessed=bytes_accessed),
        interpret=interpret,
    )(xt, m1a, m1b, m2a, m2b, bc1, bc2, w1s, b1f, w2p, b2f, w3p, b3f)
    return out[:n]


def kernel(x, w1c, b1c, w2c, b2c, wf1, bf1, wf2, bf2, wf3, bf3):
    return _forward(x, w1c, b1c, w2c, b2c, wf1, bf1, wf2, bf2, wf3, bf3)


# speedup vs baseline: 142.5037x; 4.0042x over previous
import functools

import jax
import jax.numpy as jnp
from jax.experimental import pallas as pl
from jax.experimental.pallas import tpu as pltpu


def _round_up(x, m):
    return (x + m - 1) // m * m


_W_IN1, _W_IN2 = 56, 156
_OW1, _OW2 = 26, 11
_C1, _C2 = 6, 16
_N1, _N2 = _C1 * _OW1, _C2 * _OW2
_R1, _R2 = 12, 4


def _fused_kernel(xt_ref, m1a_ref, m1b_ref, m2a_ref, m2b_ref, bc1_ref,
                  bc2_ref, w1_ref, b1_ref, w2_ref, b2_ref, w3_ref, b3_ref,
                  o_ref):
    y1 = []
    for r in range(_R1):
        acc = None
        for a in range(2):
            p = jnp.concatenate(
                [xt_ref[2 * r + a + kh] for kh in range(5)], axis=1)
            for m_ref in (m1a_ref, m1b_ref):
                z = jnp.dot(p, m_ref[...],
                            preferred_element_type=jnp.float32)
                acc = z if acc is None else jnp.maximum(acc, z)
        y1.append(jnp.maximum(acc + bc1_ref[...], 0.0))

    y2 = []
    for r in range(_R2):
        acc = None
        for a in range(2):
            p = jnp.concatenate(
                [y1[2 * r + a + kh] for kh in range(5)], axis=1)
            for m_ref in (m2a_ref, m2b_ref):
                z = jnp.dot(p, m_ref[...],
                            preferred_element_type=jnp.float32)
                acc = z if acc is None else jnp.maximum(acc, z)
        y2.append(jnp.maximum(acc + bc2_ref[...], 0.0))

    feat = jnp.concatenate(y2, axis=1)
    h = jnp.dot(feat, w1_ref[...], preferred_element_type=jnp.float32)
    h = jnp.maximum(h + b1_ref[...], 0.0)
    h = jnp.dot(h, w2_ref[...], preferred_element_type=jnp.float32)
    h = jnp.maximum(h + b2_ref[...], 0.0)
    z = jnp.dot(h, w3_ref[...], preferred_element_type=jnp.float32)
    z = z + b3_ref[...]
    z = z - jnp.max(z, axis=-1, keepdims=True)
    e = jnp.exp(z)
    o_ref[...] = (e / jnp.sum(e, axis=-1, keepdims=True))[:, :o_ref.shape[1]]


def _banded_conv_matrix(w, c_in, c_out, w_in, ow, b):
    period = w_in + 2
    blk = jnp.pad(w, ((0, 0), (0, 0), (0, 0), (0, period - 5)))
    tiled = jnp.broadcast_to(blk[:, :, :, None, :],
                             (c_out, c_in, 5, ow, period))
    flat = tiled.reshape(c_out, c_in, 5, ow * period)
    flat = jnp.pad(flat, ((0, 0),) * 3 + ((b, 0),))[..., :ow * w_in]
    t = flat.reshape(c_out, c_in, 5, ow, w_in)
    return t.transpose(2, 1, 4, 0, 3).reshape(5 * c_in * w_in, c_out * ow)


@functools.partial(jax.jit, static_argnames=("interpret",))
def _forward(x, w1c, b1c, w2c, b2c, wf1, bf1, wf2, bf2, wf3, bf3,
             interpret=False):
    n = x.shape[0]
    bt = 256 if n >= 256 else _round_up(n, 8)
    n_pad = _round_up(n, bt)
    xs = x[:, 0]
    if n_pad != n:
        xs = jnp.pad(xs, ((0, n_pad - n), (0, 0), (0, 0)))
    xt = xs.transpose(1, 0, 2)

    w1 = w1c.reshape(_C1, 1, 5, 5).astype(jnp.float32)
    w2 = w2c.astype(jnp.float32)
    m1a = _banded_conv_matrix(w1, 1, _C1, _W_IN1, _OW1, 0)
    m1b = _banded_conv_matrix(w1, 1, _C1, _W_IN1, _OW1, 1)
    m2a = _banded_conv_matrix(w2, _C1, _C2, _OW1, _OW2, 0)
    m2b = _banded_conv_matrix(w2, _C1, _C2, _OW1, _OW2, 1)
    bc1 = jnp.repeat(b1c, _OW1).reshape(1, _N1)
    bc2 = jnp.repeat(b2c, _OW2).reshape(1, _N2)

    w1s = wf1.reshape(120, _C2, _R2, _OW2).transpose(2, 1, 3, 0)
    w1s = jnp.pad(w1s.reshape(_R2 * _N2, 120), ((0, 0), (0, 8)))
    b1f = jnp.pad(bf1, (0, 8)).reshape(1, 128)
    w2p = jnp.pad(wf2.T, ((0, 8), (0, 44)))
    b2f = jnp.pad(bf2, (0, 44)).reshape(1, 128)
    n_cls = wf3.shape[0]
    w3p = jnp.pad(wf3.T, ((0, 44), (0, 128 - n_cls)))
    b3f = jnp.concatenate(
        [bf3, jnp.full((128 - n_cls,), -jnp.inf, jnp.float32)]).reshape(1, 128)

    grid = (n_pad // bt,)
    flops = 2 * n_pad * (_R1 * 4 * 280 * _N1 + _R2 * 4 * 780 * _N2
                         + 704 * 128 + 2 * 128 * 128)
    bytes_accessed = 4 * (28 * n_pad * 56 + n_pad * n_cls + 300000)

    out = pl.pallas_call(
        _fused_kernel,
        out_shape=jax.ShapeDtypeStruct((n_pad, n_cls), jnp.float32),
        grid=grid,
        in_specs=[
            pl.BlockSpec((28, bt, 56), lambda i: (0, i, 0)),
            pl.BlockSpec((280, _N1), lambda i: (0, 0)),
            pl.BlockSpec((280, _N1), lambda i: (0, 0)),
            pl.BlockSpec((780, _N2), lambda i: (0, 0)),
            pl.BlockSpec((780, _N2), lambda i: (0, 0)),
            pl.BlockSpec((1, _N1), lambda i: (0, 0)),
            pl.BlockSpec((1, _N2), lambda i: (0, 0)),
            pl.BlockSpec((_R2 * _N2, 128), lambda i: (0, 0)),
            pl.BlockSpec((1, 128), lambda i: (0, 0)),
            pl.BlockSpec((128, 128), lambda i: (0, 0)),
            pl.BlockSpec((1, 128), lambda i: (0, 0)),
            pl.BlockSpec((128, 128), lambda i: (0, 0)),
            pl.BlockSpec((1, 128), lambda i: (0, 0)),
        ],
        out_specs=pl.BlockSpec((bt, n_cls), lambda i: (i, 0)),
        compiler_params=pltpu.CompilerParams(
            dimension_semantics=("parallel",),
            vmem_limit_bytes=64 * 1024 * 1024),
        cost_estimate=pl.CostEstimate(flops=flops, transcendentals=n_pad * 128,
                                      bytes_accessed=bytes_accessed),
        interpret=interpret,
    )(xt, m1a, m1b, m2a, m2b, bc1, bc2, w1s, b1f, w2p, b2f, w3p, b3f)
    return out[:n]


def kernel(x, w1c, b1c, w2c, b2c, wf1, bf1, wf2, bf2, wf3, bf3):
    return _forward(x, w1c, b1c, w2c, b2c, wf1, bf1, wf2, bf2, wf3, bf3)
